# Initial kernel scaffold; baseline (speedup 1.0000x reference)
import jax
import jax.numpy as jnp
from jax.experimental import pallas as pl


def kernel(x1, x2, out_index, out_val, in_index, in_val, W_out, W_in, bias1, bias2):
    raise NotImplementedError("write your pallas kernel here")


# speedup vs baseline: 3.5792x; 3.5792x over previous
import functools

import jax
import jax.numpy as jnp
from jax import lax
from jax.experimental import pallas as pl
from jax.experimental.pallas import tpu as pltpu
from jax.experimental.pallas import tpu_sc as plsc

N = 10000
E = 320000
D = 128

_NS = 16
_EPT = E // _NS
_C = 80
_NCHUNK = _EPT // _C
_RPT = 624
_TAIL = N - _NS * _RPT
_ZROWS = 208


def _mm_body(x_ref, w_ref, o_ref):
    o_ref[...] = jnp.dot(x_ref[0], w_ref[0],
                         preferred_element_type=jnp.float32)[None]


def _matmul(xs, ws):
    return pl.pallas_call(
        _mm_body,
        grid=(2, 10),
        in_specs=[
            pl.BlockSpec((1, N // 10, D), lambda b, i: (b, i, 0)),
            pl.BlockSpec((1, D, D), lambda b, i: (b, 0, 0)),
        ],
        out_specs=pl.BlockSpec((1, N // 10, D), lambda b, i: (b, i, 0)),
        out_shape=jax.ShapeDtypeStruct((2, N, D), jnp.float32),
    )(xs, ws)


def _ep_body(p_ref, b_ref, o_ref):
    o_ref[...] = jnp.maximum(p_ref[...] + b_ref[0], 0.0)


def _epilogue(p2, biases):
    return pl.pallas_call(
        _ep_body,
        grid=(2, 10),
        in_specs=[
            pl.BlockSpec((1, N // 10, D), lambda b, i: (b, i, 0)),
            pl.BlockSpec((1, 1, D), lambda b, i: (b, 0, 0)),
        ],
        out_specs=pl.BlockSpec((1, N // 10, D), lambda b, i: (b, i, 0)),
        out_shape=jax.ShapeDtypeStruct((2, N, D), jnp.float32),
    )(p2, biases)


def _sc_spmm(h2, rows, cols, vals):
    mesh = plsc.VectorSubcoreMesh(core_axis_name="c", subcore_axis_name="s")

    @functools.partial(
        pl.kernel,
        mesh=mesh,
        out_type=jax.ShapeDtypeStruct((2 * N, D), jnp.float32),
        scratch_types=[
            pltpu.VMEM_SHARED((N, D), jnp.float32),
            pltpu.VMEM((_C,), jnp.int32),
            pltpu.VMEM((_C,), jnp.int32),
            pltpu.VMEM((_C,), jnp.float32),
            pltpu.VMEM((_C, D), jnp.float32),
            pltpu.VMEM((_ZROWS, D), jnp.float32),
            pltpu.SemaphoreType.DMA,
        ],
    )
    def spmm(h_hbm, row_hbm, col_hbm, val_hbm, out_hbm,
             acc, cidx, ridx, val_v, rows_v, zbuf, sem):
        c = lax.axis_index("c")
        s = lax.axis_index("s")

        zero16 = jnp.zeros((16,), jnp.float32)

        def zfill(i, carry):
            for j in range(D // 16):
                zbuf[i, pl.ds(16 * j, 16)] = zero16
            return carry

        lax.fori_loop(0, _ZROWS, zfill, 0)
        for k in range(_RPT // _ZROWS):
            pltpu.sync_copy(zbuf, acc.at[pl.ds(s * _RPT + k * _ZROWS, _ZROWS)])

        @pl.when(s == 0)
        def _zero_tail():
            pltpu.sync_copy(zbuf.at[pl.ds(0, _TAIL)],
                            acc.at[pl.ds(_NS * _RPT, _TAIL)])

        plsc.subcore_barrier()

        base = c * E + s * _EPT

        def chunk(i, carry):
            eb = base + i * _C
            pltpu.sync_copy(col_hbm.at[pl.ds(eb, _C)], cidx)
            pltpu.sync_copy(row_hbm.at[pl.ds(eb, _C)], ridx)
            pltpu.sync_copy(val_hbm.at[pl.ds(eb, _C)], val_v)
            pltpu.async_copy(h_hbm.at[cidx], rows_v, sem).wait()

            def group(g, carry2):
                e0 = g * 16
                vv = val_v[pl.ds(e0, 16)]
                for l in range(16):
                    v = vv[l]
                    for j in range(D // 16):
                        sl = pl.ds(16 * j, 16)
                        rows_v[e0 + l, sl] = rows_v[e0 + l, sl] * v
                return carry2

            lax.fori_loop(0, _C // 16, group, 0)
            pltpu.sync_copy(rows_v, acc.at[ridx], add=True)
            return carry

        lax.fori_loop(0, _NCHUNK, chunk, 0)
        plsc.subcore_barrier()

        r0 = s * _RPT
        pltpu.sync_copy(acc.at[pl.ds(r0, _RPT)],
                        out_hbm.at[pl.ds(c * N + r0, _RPT)])

        @pl.when(s == 0)
        def _write_tail():
            pltpu.sync_copy(acc.at[pl.ds(_NS * _RPT, _TAIL)],
                            out_hbm.at[pl.ds(c * N + _NS * _RPT, _TAIL)])

    return spmm(h2, rows, cols, vals)


def kernel(x1, x2, out_index, out_val, in_index, in_val,
           W_out, W_in, bias1, bias2):
    xs = jnp.stack([x1, x2])
    ws = jnp.stack([W_out, W_in])
    h2 = _matmul(xs, ws).reshape(2 * N, D)

    rows = jnp.concatenate([out_index[0], in_index[0]]).astype(jnp.int32)
    cols = jnp.concatenate([out_index[1], in_index[1] + N]).astype(jnp.int32)
    vals = jnp.concatenate([out_val, in_val])

    p2 = _sc_spmm(h2, rows, cols, vals).reshape(2, N, D)

    biases = jnp.stack([bias1, bias2]).reshape(2, 1, D)
    outs = _epilogue(p2, biases)
    return (outs[0], outs[1])


# speedup vs baseline: 6.3833x; 1.7834x over previous
import functools

import jax
import jax.numpy as jnp
from jax import lax
from jax.experimental import pallas as pl
from jax.experimental.pallas import tpu as pltpu
from jax.experimental.pallas import tpu_sc as plsc

N = 10000
E = 320000
D = 128

_NS = 16
_C = 112
_NCH = 180
_EPT = _NCH * _C
_PE = _NS * _EPT
_NR2 = 2 * _PE // _C
_RPT = 624
_TAIL = N - _NS * _RPT
_ZR = 16


def _mm_body(x_ref, w_ref, o_ref):
    o_ref[...] = jnp.dot(x_ref[0], w_ref[0],
                         preferred_element_type=jnp.float32)[None]


def _matmul(xs, ws):
    return pl.pallas_call(
        _mm_body,
        grid=(2, 10),
        in_specs=[
            pl.BlockSpec((1, N // 10, D), lambda b, i: (b, i, 0)),
            pl.BlockSpec((1, D, D), lambda b, i: (b, 0, 0)),
        ],
        out_specs=pl.BlockSpec((1, N // 10, D), lambda b, i: (b, i, 0)),
        out_shape=jax.ShapeDtypeStruct((2, N, D), jnp.float32),
    )(xs, ws)


def _ep_body(p_ref, b_ref, o_ref):
    o_ref[...] = jnp.maximum(p_ref[...] + b_ref[0], 0.0)


def _epilogue(p2, biases):
    return pl.pallas_call(
        _ep_body,
        grid=(2, 10),
        in_specs=[
            pl.BlockSpec((1, N // 10, D), lambda b, i: (b, i, 0)),
            pl.BlockSpec((1, 1, D), lambda b, i: (b, 0, 0)),
        ],
        out_specs=pl.BlockSpec((1, N // 10, D), lambda b, i: (b, i, 0)),
        out_shape=jax.ShapeDtypeStruct((2, N, D), jnp.float32),
    )(p2, biases)


def _sc_spmm(h2, edata, vdata):
    mesh = plsc.VectorSubcoreMesh(core_axis_name="c", subcore_axis_name="s")

    @functools.partial(
        pl.kernel,
        mesh=mesh,
        out_type=jax.ShapeDtypeStruct((2 * N, D), jnp.float32),
        scratch_types=[
            pltpu.VMEM_SHARED((N, D), jnp.float32),
            pltpu.VMEM((_C, D), jnp.float32),
            pltpu.VMEM((_C, D), jnp.float32),
            pltpu.VMEM((_C, D), jnp.float32),
            pltpu.VMEM((2, _C), jnp.int32),
            pltpu.VMEM((2, _C), jnp.int32),
            pltpu.VMEM((2, _C), jnp.int32),
            pltpu.VMEM((2, _C), jnp.int32),
            pltpu.VMEM((2, _C), jnp.int32),
            pltpu.VMEM((2, _C), jnp.int32),
            pltpu.VMEM((_C,), jnp.float32),
            pltpu.VMEM((_C,), jnp.float32),
            pltpu.VMEM((_C,), jnp.float32),
            pltpu.VMEM((_C,), jnp.float32),
            pltpu.VMEM((_C,), jnp.float32),
            pltpu.VMEM((_C,), jnp.float32),
            pltpu.VMEM((_ZR, D), jnp.float32),
            pltpu.SemaphoreType.DMA,
            pltpu.SemaphoreType.DMA,
            pltpu.SemaphoreType.DMA,
            pltpu.SemaphoreType.DMA,
            pltpu.SemaphoreType.DMA,
            pltpu.SemaphoreType.DMA,
            pltpu.SemaphoreType.DMA,
            pltpu.SemaphoreType.DMA,
            pltpu.SemaphoreType.DMA,
            pltpu.SemaphoreType.DMA,
            pltpu.SemaphoreType.DMA,
            pltpu.SemaphoreType.DMA,
            pltpu.SemaphoreType.DMA,
            pltpu.SemaphoreType.DMA,
            pltpu.SemaphoreType.DMA,
            pltpu.SemaphoreType.DMA,
            pltpu.SemaphoreType.DMA,
            pltpu.SemaphoreType.DMA,
        ],
    )
    def spmm(h_hbm, ed_hbm, vd_hbm, out_hbm,
             acc, rv0, rv1, rv2, eb0, eb1, eb2, eb3, eb4, eb5,
             vb0, vb1, vb2, vb3, vb4, vb5, zbuf,
             gs0, gs1, gs2, ss0, ss1, ss2,
             es0, es1, es2, es3, es4, es5,
             vs0, vs1, vs2, vs3, vs4, vs5):
        c = lax.axis_index("c")
        s = lax.axis_index("s")
        rvs = (rv0, rv1, rv2)
        ebs = (eb0, eb1, eb2, eb3, eb4, eb5)
        vbs = (vb0, vb1, vb2, vb3, vb4, vb5)
        gss = (gs0, gs1, gs2)
        sss = (ss0, ss1, ss2)
        ess = (es0, es1, es2, es3, es4, es5)
        vss = (vs0, vs1, vs2, vs3, vs4, vs5)
        tb = c * (_NR2 // 2) + s * _NCH

        def estart(i, q):
            pltpu.async_copy(ed_hbm.at[tb + i], ebs[q], ess[q])
            pltpu.async_copy(vd_hbm.at[tb + i], vbs[q], vss[q])

        def ewait(q):
            pltpu.make_async_copy(ed_hbm.at[0], ebs[q], ess[q]).wait()
            pltpu.make_async_copy(vd_hbm.at[0], vbs[q], vss[q]).wait()

        def gstart(q, b):
            pltpu.async_copy(h_hbm.at[ebs[q].at[0]], rvs[b], gss[b])

        def gwait(b):
            pltpu.make_async_copy(h_hbm.at[ebs[0].at[0]], rvs[b],
                                  gss[b]).wait()

        def sstart(q, b):
            pltpu.async_copy(rvs[b], acc.at[ebs[q].at[1]], sss[b], add=True)

        def swait(b):
            pltpu.make_async_copy(rvs[b], acc.at[ebs[0].at[1]],
                                  sss[b]).wait()

        zero16 = jnp.zeros((16,), jnp.float32)

        def zfill(i, carry):
            for j in range(D // 16):
                zbuf[i, pl.ds(16 * j, 16)] = zero16
            return carry

        lax.fori_loop(0, _ZR, zfill, 0)
        for k in range(_RPT // _ZR):
            pltpu.sync_copy(zbuf, acc.at[pl.ds(s * _RPT + k * _ZR, _ZR)])

        @pl.when(s == 0)
        def _zero_tail():
            pltpu.sync_copy(zbuf, acc.at[pl.ds(_NS * _RPT, _TAIL)])

        estart(0, 0)
        estart(1, 1)
        estart(2, 2)
        ewait(0)
        gstart(0, 0)
        ewait(1)
        gstart(1, 1)
        plsc.subcore_barrier()

        def six(t, carry):
            for k in range(6):
                i = 6 * t + k
                b = k % 3
                q = k
                gwait(b)
                rv = rvs[b]

                def group(g, carry2, _rv=rv, _q=q):
                    e0 = g * 16
                    vv = vbs[_q][pl.ds(e0, 16)]
                    for l in range(16):
                        v = vv[l]
                        for j in range(D // 16):
                            sl = pl.ds(16 * j, 16)
                            _rv[e0 + l, sl] = _rv[e0 + l, sl] * v
                    return carry2

                lax.fori_loop(0, _C // 16, group, 0)

                @pl.when(i > 0)
                def _sw(_b=(b + 2) % 3):
                    swait(_b)

                @pl.when(i + 2 < _NCH)
                def _eg(_i=i, _q=(q + 2) % 6, _b=(b + 2) % 3):
                    ewait(_q)
                    gstart(_q, _b)

                @pl.when(i + 3 < _NCH)
                def _es(_i=i, _q=(q + 3) % 6):
                    estart(_i + 3, _q)

                sstart(q, b)
            return carry

        lax.fori_loop(0, _NCH // 6, six, 0)
        swait((_NCH - 1) % 3)
        plsc.subcore_barrier()

        r0 = s * _RPT
        pltpu.sync_copy(acc.at[pl.ds(r0, _RPT)],
                        out_hbm.at[pl.ds(c * N + r0, _RPT)])

        @pl.when(s == 0)
        def _write_tail():
            pltpu.sync_copy(acc.at[pl.ds(_NS * _RPT, _TAIL)],
                            out_hbm.at[pl.ds(c * N + _NS * _RPT, _TAIL)])

    return spmm(h2, edata, vdata)


def kernel(x1, x2, out_index, out_val, in_index, in_val,
           W_out, W_in, bias1, bias2):
    xs = jnp.stack([x1, x2])
    ws = jnp.stack([W_out, W_in])
    h2 = _matmul(xs, ws).reshape(2 * N, D)

    npad = _PE - E
    padi = jnp.zeros((npad,), jnp.int32)
    padf = jnp.zeros((npad,), jnp.float32)
    r0 = out_index[0].astype(jnp.int32)
    c0 = out_index[1].astype(jnp.int32)
    r1 = in_index[0].astype(jnp.int32)
    c1 = in_index[1].astype(jnp.int32)
    rows2 = jnp.concatenate([r0, padi, r1, padi]).reshape(_NR2, _C)
    cols2 = jnp.concatenate([c0, padi, c1 + N, padi]).reshape(_NR2, _C)
    vdata = jnp.concatenate([out_val, padf, in_val, padf]).reshape(_NR2, _C)
    edata = jnp.stack([cols2, rows2], axis=1)

    p2 = _sc_spmm(h2, edata, vdata).reshape(2, N, D)

    biases = jnp.stack([bias1, bias2]).reshape(2, 1, D)
    outs = _epilogue(p2, biases)
    return (outs[0], outs[1])
